# baseline (speedup 1.0000x reference)
import functools

import jax
import jax.numpy as jnp
from jax import lax
from jax.experimental import pallas as pl
from jax.experimental.pallas import tpu as pltpu
from jax.experimental.pallas import tpu_sc as plsc

D = 1024
E = 8
FF = 4096
T = 2048
TB = 512
BLK = 128
NB = 40
NPAD = NB * BLK
NF2 = 2
FT2 = FF // NF2

NC = 2
NS = 16
NW = NC * NS
RPT = NPAD // NW
GCH = 40
TPT = T // NW
CCH = 16


def _gelu_exact(v):
    return v * 0.5 * (1.0 + lax.erf(v * 0.7071067811865476))


def _prep_kernel(x_ref, gw_ref, gb_ref, gamma_ref, beta_ref, xn_ref, lg_ref):
    xb = x_ref[...]
    mu = jnp.mean(xb, axis=-1, keepdims=True)
    var = jnp.mean((xb - mu) ** 2, axis=-1, keepdims=True)
    xn = (xb - mu) / jnp.sqrt(var + 1e-5) * gamma_ref[...] + beta_ref[...]
    xn_ref[...] = xn
    lg_ref[...] = jnp.dot(xn, gw_ref[...],
                          preferred_element_type=jnp.float32) + gb_ref[...]


def _route_kernel(lg_ref, mi_ref, mf_ref, be_ref):
    logits = lg_ref[...]
    m = jnp.max(logits, axis=-1, keepdims=True)
    ex = jnp.exp(logits - m)
    probs = ex / jnp.sum(ex, axis=-1, keepdims=True)
    lane = lax.broadcasted_iota(jnp.int32, (T, E), 1)
    m0 = jnp.max(probs, axis=-1, keepdims=True)
    e0 = jnp.min(jnp.where(probs == m0, lane, E), axis=-1, keepdims=True)
    probs1 = jnp.where(lane == e0, -1.0, probs)
    m1 = jnp.max(probs1, axis=-1, keepdims=True)
    e1 = jnp.min(jnp.where(probs1 == m1, lane, E), axis=-1, keepdims=True)
    denom = m0 + m1 + 1e-8
    w0 = m0 / denom
    w1 = m1 / denom

    oh0 = (lane == e0).astype(jnp.float32)
    oh1 = (lane == e1).astype(jnp.float32)
    a01 = oh0 + oh1
    r = lax.broadcasted_iota(jnp.int32, (T, T), 0)
    c = lax.broadcasted_iota(jnp.int32, (T, T), 1)
    tril = (r > c).astype(jnp.bfloat16)
    cnt = jnp.dot(tril, a01.astype(jnp.bfloat16),
                  preferred_element_type=jnp.float32)

    counts = jnp.sum(a01, axis=0, keepdims=True)
    rounded = jnp.ceil(counts / BLK) * BLK
    s = rounded
    for sh in (1, 2, 4):
        s = s + jnp.concatenate(
            [jnp.zeros((1, sh), jnp.float32), s[:, :E - sh]], axis=1)
    pstart = s - rounded

    pos = pstart + cnt
    pos0 = jnp.sum(jnp.where(lane == e0, pos, 0.0), axis=-1, keepdims=True)
    pos1 = jnp.sum(jnp.where(lane == e1, pos, 0.0), axis=-1, keepdims=True)
    mi_ref[...] = jnp.where(lane == 0, pos0.astype(jnp.int32),
                            jnp.where(lane == 1, pos1.astype(jnp.int32), 0))
    mf_ref[...] = jnp.where(lane == 0, w0, jnp.where(lane == 1, w1, 0.0))

    pend = pstart + rounded
    bi = (lax.broadcasted_iota(jnp.int32, (NB, 1), 0) * BLK
          ).astype(jnp.float32)
    be = jnp.sum((pend <= bi).astype(jnp.int32), axis=-1, keepdims=True)
    be_ref[...] = jnp.minimum(be, E - 1)


def _dispatch_kernel(xn_hbm, p0_hbm, p1_hbm, w0_hbm, w1_hbm,
                     xs_hbm, ws_hbm,
                     pidx_v, vi_v, vf_v, zi_v, zf_v, idx_v, rows_v,
                     sh_inv, sh_ws, sem):
    cid = lax.axis_index("c")
    sid = lax.axis_index("s")
    wid = sid * NC + cid
    TPS = T // NS
    ZS = NPAD // NS

    def zbody(i, _):
        zi_v[pl.ds(i * 16, 16)] = jnp.zeros((16,), jnp.int32)
        zf_v[pl.ds(i * 16, 16)] = jnp.zeros((16,), jnp.float32)
        return 0
    lax.fori_loop(0, ZS // 16, zbody, 0)
    pltpu.sync_copy(zi_v, sh_inv.at[pl.ds(sid * ZS, ZS)])
    pltpu.sync_copy(zf_v, sh_ws.at[pl.ds(sid * ZS, ZS)])
    plsc.subcore_barrier()

    base_t = sid * TPS
    iota = lax.iota(jnp.int32, 16)

    def fill_body(i, _):
        vi_v[pl.ds(i * 16, 16)] = base_t + i * 16 + iota
        return 0
    lax.fori_loop(0, TPS // 16, fill_body, 0)

    for (p_hbm, w_hbm) in ((p0_hbm, w0_hbm), (p1_hbm, w1_hbm)):
        pltpu.sync_copy(p_hbm.at[pl.ds(base_t, TPS)], pidx_v)
        pltpu.sync_copy(w_hbm.at[pl.ds(base_t, TPS)], vf_v)
        pltpu.sync_copy(vi_v, sh_inv.at[pidx_v], add=True)
        pltpu.sync_copy(vf_v, sh_ws.at[pidx_v], add=True)
    plsc.subcore_barrier()

    @pl.when(wid == 0)
    def _():
        pltpu.sync_copy(sh_ws, ws_hbm)

    base = wid * RPT
    pltpu.sync_copy(sh_inv.at[pl.ds(base, RPT)], idx_v)
    for ch in range(RPT // GCH):
        pltpu.async_copy(xn_hbm.at[idx_v.at[pl.ds(ch * GCH, GCH)]],
                         rows_v, sem).wait()
        pltpu.sync_copy(rows_v, xs_hbm.at[pl.ds(base + ch * GCH, GCH)])


def _ffn_kernel(be_ref, xs_ref, w1_ref, b1_ref, w2_ref, b2_ref, ws_ref,
                out_ref):
    f = pl.program_id(0)
    xb = xs_ref[...].astype(jnp.bfloat16)
    h = jnp.dot(xb, w1_ref[0].astype(jnp.bfloat16),
                preferred_element_type=jnp.float32) + b1_ref[0]
    h = _gelu_exact(h).astype(jnp.bfloat16)
    y = jnp.dot(h, w2_ref[0].astype(jnp.bfloat16),
                preferred_element_type=jnp.float32)
    y = y + jnp.where(f == 0, 1.0, 0.0) * b2_ref[0]
    out_ref[0] = ws_ref[0] * y


def _combine_kernel(x_hbm, y_hbm, p0_hbm, p1_hbm, out_hbm,
                    p0_v, p1_v, q0_v, q1_v,
                    y00_v, y01_v, y10_v, y11_v, x_v, o_v, sems):
    cid = lax.axis_index("c")
    sid = lax.axis_index("s")
    wid = sid * NC + cid
    base = wid * TPT

    pltpu.sync_copy(p0_hbm.at[pl.ds(base, TPT)], p0_v)
    pltpu.sync_copy(p1_hbm.at[pl.ds(base, TPT)], p1_v)

    def off_body(i, _):
        q0_v[pl.ds(i * 16, 16)] = p0_v[pl.ds(i * 16, 16)] + NPAD
        q1_v[pl.ds(i * 16, 16)] = p1_v[pl.ds(i * 16, 16)] + NPAD
        return 0
    lax.fori_loop(0, TPT // 16, off_body, 0)

    for ch in range(TPT // CCH):
        o = ch * CCH
        cp = [
            pltpu.async_copy(y_hbm.at[p0_v.at[pl.ds(o, CCH)]], y00_v, sems[0]),
            pltpu.async_copy(y_hbm.at[q0_v.at[pl.ds(o, CCH)]], y01_v, sems[1]),
            pltpu.async_copy(y_hbm.at[p1_v.at[pl.ds(o, CCH)]], y10_v, sems[2]),
            pltpu.async_copy(y_hbm.at[q1_v.at[pl.ds(o, CCH)]], y11_v, sems[3]),
        ]
        pltpu.sync_copy(x_hbm.at[pl.ds(base + o, CCH)], x_v)
        for c in cp:
            c.wait()

        def sum_body(rr, _):
            for v in range(D // 16):
                sl = pl.ds(v * 16, 16)
                o_v[rr, sl] = (x_v[rr, sl] + (y00_v[rr, sl] + y01_v[rr, sl])
                               + (y10_v[rr, sl] + y11_v[rr, sl]))
            return 0
        lax.fori_loop(0, CCH, sum_body, 0)
        pltpu.sync_copy(o_v, out_hbm.at[pl.ds(base + o, CCH)])


def kernel(x, gate_W, gate_b, W1, b1, W2, b2, gamma, beta):
    b, s, d = x.shape
    flat = x.reshape(-1, d)

    xn, logits = pl.pallas_call(
        _prep_kernel,
        grid=(T // TB,),
        in_specs=[
            pl.BlockSpec((TB, D), lambda i: (i, 0)),
            pl.BlockSpec((D, E), lambda i: (0, 0)),
            pl.BlockSpec((E,), lambda i: (0,)),
            pl.BlockSpec((D,), lambda i: (0,)),
            pl.BlockSpec((D,), lambda i: (0,)),
        ],
        out_specs=[
            pl.BlockSpec((TB, D), lambda i: (i, 0)),
            pl.BlockSpec((TB, E), lambda i: (i, 0)),
        ],
        out_shape=[
            jax.ShapeDtypeStruct((T, D), jnp.float32),
            jax.ShapeDtypeStruct((T, E), jnp.float32),
        ],
    )(flat, gate_W, gate_b, gamma, beta)

    mi, mf, be = pl.pallas_call(
        _route_kernel,
        out_shape=[
            jax.ShapeDtypeStruct((T, E), jnp.int32),
            jax.ShapeDtypeStruct((T, E), jnp.float32),
            jax.ShapeDtypeStruct((NB, 1), jnp.int32),
        ],
    )(logits)

    pos0 = mi[:, 0]
    pos1 = mi[:, 1]
    w0 = mf[:, 0]
    w1 = mf[:, 1]
    blk_e = be.reshape(NB)

    mesh = plsc.VectorSubcoreMesh(core_axis_name="c", subcore_axis_name="s")
    xs, wsort = pl.kernel(
        _dispatch_kernel,
        out_type=[
            jax.ShapeDtypeStruct((NPAD, D), jnp.float32),
            jax.ShapeDtypeStruct((NPAD,), jnp.float32),
        ],
        mesh=mesh,
        scratch_types=[
            pltpu.VMEM((T // NS,), jnp.int32),
            pltpu.VMEM((T // NS,), jnp.int32),
            pltpu.VMEM((T // NS,), jnp.float32),
            pltpu.VMEM((NPAD // NS,), jnp.int32),
            pltpu.VMEM((NPAD // NS,), jnp.float32),
            pltpu.VMEM((RPT,), jnp.int32),
            pltpu.VMEM((GCH, D), jnp.float32),
            pltpu.VMEM_SHARED((NPAD,), jnp.int32),
            pltpu.VMEM_SHARED((NPAD,), jnp.float32),
            pltpu.SemaphoreType.DMA,
        ],
    )(xn, pos0, pos1, w0, w1)

    grid_spec = pltpu.PrefetchScalarGridSpec(
        num_scalar_prefetch=1,
        grid=(NF2, NB),
        in_specs=[
            pl.BlockSpec((BLK, D), lambda f, b, be_r: (b, 0)),
            pl.BlockSpec((1, D, FT2), lambda f, b, be_r: (be_r[b], 0, f)),
            pl.BlockSpec((1, 1, FT2), lambda f, b, be_r: (be_r[b] * NF2 + f,
                                                          0, 0)),
            pl.BlockSpec((1, FT2, D), lambda f, b, be_r: (be_r[b], f, 0)),
            pl.BlockSpec((1, 1, D), lambda f, b, be_r: (be_r[b], 0, 0)),
            pl.BlockSpec((1, BLK, 1), lambda f, b, be_r: (b, 0, 0)),
        ],
        out_specs=pl.BlockSpec((1, BLK, D), lambda f, b, be_r: (f, b, 0)),
    )
    yp = pl.pallas_call(
        _ffn_kernel,
        grid_spec=grid_spec,
        out_shape=jax.ShapeDtypeStruct((NF2, NPAD, D), jnp.float32),
        compiler_params=pltpu.CompilerParams(
            dimension_semantics=("arbitrary", "arbitrary"),
        ),
    )(blk_e, xs, W1, b1.reshape(E * NF2, 1, FT2), W2, b2.reshape(E, 1, D),
      wsort.reshape(NB, BLK, 1))

    out = pl.kernel(
        _combine_kernel,
        out_type=jax.ShapeDtypeStruct((T, D), jnp.float32),
        mesh=mesh,
        scratch_types=[
            pltpu.VMEM((TPT,), jnp.int32),
            pltpu.VMEM((TPT,), jnp.int32),
            pltpu.VMEM((TPT,), jnp.int32),
            pltpu.VMEM((TPT,), jnp.int32),
            pltpu.VMEM((CCH, D), jnp.float32),
            pltpu.VMEM((CCH, D), jnp.float32),
            pltpu.VMEM((CCH, D), jnp.float32),
            pltpu.VMEM((CCH, D), jnp.float32),
            pltpu.VMEM((CCH, D), jnp.float32),
            pltpu.VMEM((CCH, D), jnp.float32),
            [pltpu.SemaphoreType.DMA] * 4,
        ],
    )(flat, yp.reshape(NF2 * NPAD, D), pos0, pos1)

    return out.reshape(b, s, d)
